# baseline (speedup 1.0000x reference)
import functools

import jax
import jax.numpy as jnp
from jax import lax
from jax.experimental import pallas as pl

_S = 7
_F32 = jnp.float32
_INTERPRET = False


def _pcall(body, **kw):
    return pl.pallas_call(body, interpret=_INTERPRET, **kw)


_EXACT = lax.Precision.HIGHEST


def _gdot(oh, vals):
    hi = vals.astype(jnp.bfloat16)
    lo = (vals - hi.astype(_F32)).astype(jnp.bfloat16)
    ohb = oh.astype(jnp.bfloat16)
    return (jnp.dot(ohb, hi, preferred_element_type=_F32) +
            jnp.dot(ohb, lo, preferred_element_type=_F32))


def _rowsum_sq_bcast(x, t_rows):
    s2 = jnp.sum(x * x, axis=1, keepdims=True)
    ones = jnp.ones((t_rows, 1), _F32)
    return lax.dot_general(ones, s2, (((1,), (1,)), ((), ())),
                           preferred_element_type=_F32, precision=_EXACT)


def _inner_tn(a, b):
    return lax.dot_general(a.astype(jnp.bfloat16), b.astype(jnp.bfloat16),
                           (((1,), (1,)), ((), ())),
                           preferred_element_type=_F32)


def _knn_body(tgt_ref, src_ref, out_ref, *, k):
    tgt = tgt_ref[0]
    src = src_ref[0]
    t_rows = tgt.shape[0]
    v = src.shape[0]
    inner = _inner_tn(tgt, src)
    t2 = jnp.sum(tgt * tgt, axis=1, keepdims=True)
    s2bc = _rowsum_sq_bcast(src, t_rows)
    d = (-2.0 * inner + s2bc) + t2
    viota = lax.broadcasted_iota(jnp.int32, (t_rows, v), 1)
    cur = d
    cols = []
    for j in range(k + 1):
        m = jnp.min(cur, axis=1, keepdims=True)
        am = jnp.min(jnp.where(cur <= m, viota, v), axis=1, keepdims=True)
        cols.append(am)
        cur = jnp.where(viota == am, jnp.inf, cur)
    out_ref[0] = jnp.concatenate(cols[1:], axis=1)


def _knn(tgt, src, k):
    bs, t_rows, _ = tgt.shape
    v = src.shape[1]
    return _pcall(
        functools.partial(_knn_body, k=k),
        grid=(bs,),
        in_specs=[pl.BlockSpec((1, t_rows, 3), lambda b: (b, 0, 0)),
                  pl.BlockSpec((1, v, 3), lambda b: (b, 0, 0))],
        out_specs=pl.BlockSpec((1, t_rows, k), lambda b: (b, 0, 0)),
        out_shape=jax.ShapeDtypeStruct((bs, t_rows, k), jnp.int32),
    )(tgt, src)


def _support_sum(maxed, cout):
    sc = maxed.shape[1]
    r = lax.broadcasted_iota(jnp.int32, (sc, cout), 0)
    c = lax.broadcasted_iota(jnp.int32, (sc, cout), 1)
    smat = (jnp.bitwise_and(r, cout - 1) == c).astype(jnp.bfloat16)
    hi = maxed.astype(jnp.bfloat16)
    lo = (maxed - hi.astype(_F32)).astype(jnp.bfloat16)
    return (jnp.dot(hi, smat, preferred_element_type=_F32) +
            jnp.dot(lo, smat, preferred_element_type=_F32))


def _normed_dirs(dirs):
    nd = jnp.sqrt(jnp.sum(dirs * dirs, axis=0, keepdims=True))
    return dirs / jnp.maximum(nd, 1e-12)


def _neighbor_theta(verts, vt, nb, sdn, j, viota):
    colj = nb[:, j:j + 1]
    oh = (viota == colj).astype(_F32)
    nbrs = jnp.dot(oh, verts, preferred_element_type=_F32, precision=_EXACT)
    diff = nbrs - vt
    dn = jnp.sqrt(jnp.sum(diff * diff, axis=1, keepdims=True))
    dirn = diff / jnp.maximum(dn, 1e-12)
    theta = jnp.maximum(
        jnp.dot(dirn.astype(jnp.bfloat16), sdn.astype(jnp.bfloat16),
                preferred_element_type=_F32), 0.0)
    return oh, theta


def _surface_body(verts_ref, vt_ref, nb_ref, dirs_ref, out_ref, *, n, cout):
    verts = verts_ref[0]
    vt = vt_ref[0]
    nb = nb_ref[0]
    sdn = _normed_dirs(dirs_ref[...])
    t_rows = vt.shape[0]
    v = verts.shape[0]
    viota = lax.broadcasted_iota(jnp.int32, (t_rows, v), 1)
    macc = jnp.full((t_rows, _S * cout), -jnp.inf, _F32)
    for j in range(n):
        _, theta = _neighbor_theta(verts, vt, nb, sdn, j, viota)
        macc = jnp.maximum(macc, theta)
    out_ref[0] = jnp.maximum(_support_sum(macc, cout), 0.0)


def _layer_body(verts_ref, vt_ref, nb_ref, dirs_ref, fc_ref, fs_ref, out_ref,
                *, n, cout):
    verts = verts_ref[0]
    vt = vt_ref[0]
    nb = nb_ref[0]
    sdn = _normed_dirs(dirs_ref[...])
    fs_full = fs_ref[0]
    t_rows = vt.shape[0]
    v = verts.shape[0]
    viota = lax.broadcasted_iota(jnp.int32, (t_rows, v), 1)
    macc = jnp.full((t_rows, _S * cout), -jnp.inf, _F32)
    for j in range(n):
        oh, theta = _neighbor_theta(verts, vt, nb, sdn, j, viota)
        fs = _gdot(oh, fs_full)
        macc = jnp.maximum(macc, theta * fs)
    out_ref[0] = fc_ref[0] + _support_sum(macc, cout)


def _conv_surface(nb, vertices, dirs, cout, tile):
    bs, v, n = nb.shape
    nt = v // tile
    return _pcall(
        functools.partial(_surface_body, n=n, cout=cout),
        grid=(bs, nt),
        in_specs=[pl.BlockSpec((1, v, 3), lambda b, t: (b, 0, 0)),
                  pl.BlockSpec((1, tile, 3), lambda b, t: (b, t, 0)),
                  pl.BlockSpec((1, tile, n), lambda b, t: (b, t, 0)),
                  pl.BlockSpec((3, _S * cout), lambda b, t: (0, 0))],
        out_specs=pl.BlockSpec((1, tile, cout), lambda b, t: (b, t, 0)),
        out_shape=jax.ShapeDtypeStruct((bs, v, cout), _F32),
    )(vertices, vertices, nb, dirs)


def _conv_layer(nb, vertices, fc, fs, dirs, cout, tile):
    bs, v, n = nb.shape
    nt = v // tile
    sc = _S * cout
    return _pcall(
        functools.partial(_layer_body, n=n, cout=cout),
        grid=(bs, nt),
        in_specs=[pl.BlockSpec((1, v, 3), lambda b, t: (b, 0, 0)),
                  pl.BlockSpec((1, tile, 3), lambda b, t: (b, t, 0)),
                  pl.BlockSpec((1, tile, n), lambda b, t: (b, t, 0)),
                  pl.BlockSpec((3, sc), lambda b, t: (0, 0)),
                  pl.BlockSpec((1, tile, cout), lambda b, t: (b, t, 0)),
                  pl.BlockSpec((1, v, sc), lambda b, t: (b, 0, 0))],
        out_specs=pl.BlockSpec((1, tile, cout), lambda b, t: (b, t, 0)),
        out_shape=jax.ShapeDtypeStruct((bs, v, cout), _F32),
    )(vertices, vertices, nb, dirs, fc, fs)


def _mm_body(x_ref, w_ref, b_ref, o_ref, *, relu):
    y = jnp.dot(x_ref[0].astype(jnp.bfloat16),
                w_ref[...].astype(jnp.bfloat16),
                preferred_element_type=_F32) + b_ref[...]
    if relu:
        y = jnp.maximum(y, 0.0)
    o_ref[0] = y


def _mm(x, w, b, relu=False):
    bs, m, k = x.shape
    n = w.shape[1]
    return _pcall(
        functools.partial(_mm_body, relu=relu),
        grid=(bs,),
        in_specs=[pl.BlockSpec((1, m, k), lambda b_: (b_, 0, 0)),
                  pl.BlockSpec((k, n), lambda b_: (0, 0)),
                  pl.BlockSpec((1, n), lambda b_: (0, 0))],
        out_specs=pl.BlockSpec((1, m, n), lambda b_: (b_, 0, 0)),
        out_shape=jax.ShapeDtypeStruct((bs, m, n), _F32),
    )(x, w, b[None, :])


def _bn_body(x_ref, g_ref, b_ref, o_ref, *, n_rows, relu):
    x = x_ref[...]
    m = jnp.sum(x, axis=0, keepdims=True) / n_rows
    xc = x - m
    var = jnp.sum(xc * xc, axis=0, keepdims=True) / n_rows
    y = g_ref[...] * xc / jnp.sqrt(var + 1e-5) + b_ref[...]
    if relu:
        y = jnp.maximum(y, 0.0)
    o_ref[...] = y


def _bn(x, g, b, relu):
    bs, v, c = x.shape
    x2 = x.reshape(bs * v, c)
    y = _pcall(
        functools.partial(_bn_body, n_rows=float(bs * v), relu=relu),
        out_shape=jax.ShapeDtypeStruct((bs * v, c), _F32),
    )(x2, g[None, :], b[None, :])
    return y.reshape(bs, v, c)


def _pool_body(tgt_ref, src_ref, fm_ref, out_ref, *, k):
    tgt = tgt_ref[0]
    src = src_ref[0]
    fm = fm_ref[0]
    t_rows = tgt.shape[0]
    v = src.shape[0]
    inner = _inner_tn(tgt, src)
    t2 = jnp.sum(tgt * tgt, axis=1, keepdims=True)
    s2bc = _rowsum_sq_bcast(src, t_rows)
    d = (-2.0 * inner + s2bc) + t2
    viota = lax.broadcasted_iota(jnp.int32, (t_rows, v), 1)
    cur = d
    pooled = jnp.full((t_rows, fm.shape[1]), -jnp.inf, _F32)
    for j in range(k + 1):
        m = jnp.min(cur, axis=1, keepdims=True)
        am = jnp.min(jnp.where(cur <= m, viota, v), axis=1, keepdims=True)
        if j > 0:
            oh = (viota == am).astype(_F32)
            g = _gdot(oh, fm)
            pooled = jnp.maximum(pooled, g)
        cur = jnp.where(viota == am, jnp.inf, cur)
    out_ref[0] = pooled


def _pool(tgt, src, fm, k):
    bs, t_rows, _ = tgt.shape
    v = src.shape[1]
    c = fm.shape[2]
    return _pcall(
        functools.partial(_pool_body, k=k),
        grid=(bs,),
        in_specs=[pl.BlockSpec((1, t_rows, 3), lambda b: (b, 0, 0)),
                  pl.BlockSpec((1, v, 3), lambda b: (b, 0, 0)),
                  pl.BlockSpec((1, v, c), lambda b: (b, 0, 0))],
        out_specs=pl.BlockSpec((1, t_rows, c), lambda b: (b, 0, 0)),
        out_shape=jax.ShapeDtypeStruct((bs, t_rows, c), _F32),
    )(tgt, src, fm)


def _nearest_oh(tgt, src):
    t_rows = tgt.shape[0]
    p = src.shape[0]
    inner = _inner_tn(tgt, src)
    t2 = jnp.sum(tgt * tgt, axis=1, keepdims=True)
    s2bc = _rowsum_sq_bcast(src, t_rows)
    d = (s2bc + t2) - 2.0 * inner
    piota = lax.broadcasted_iota(jnp.int32, (t_rows, p), 1)
    m = jnp.min(d, axis=1, keepdims=True)
    am = jnp.min(jnp.where(d <= m, piota, p), axis=1, keepdims=True)
    return (piota == am).astype(_F32)


def _near2_body(tgt_ref, src_ref, fa_ref, fb_ref, oa_ref, ob_ref):
    oh = _nearest_oh(tgt_ref[0], src_ref[0])
    oa_ref[0] = _gdot(oh, fa_ref[0])
    ob_ref[0] = _gdot(oh, fb_ref[0])


def _near1_body(tgt_ref, src_ref, fa_ref, oa_ref):
    oh = _nearest_oh(tgt_ref[0], src_ref[0])
    oa_ref[0] = _gdot(oh, fa_ref[0])


def _near2(tgt, src, fa, fb):
    bs, t_rows, _ = tgt.shape
    p = src.shape[1]
    ca, cb = fa.shape[2], fb.shape[2]
    return _pcall(
        _near2_body,
        grid=(bs,),
        in_specs=[pl.BlockSpec((1, t_rows, 3), lambda b: (b, 0, 0)),
                  pl.BlockSpec((1, p, 3), lambda b: (b, 0, 0)),
                  pl.BlockSpec((1, p, ca), lambda b: (b, 0, 0)),
                  pl.BlockSpec((1, p, cb), lambda b: (b, 0, 0))],
        out_specs=[pl.BlockSpec((1, t_rows, ca), lambda b: (b, 0, 0)),
                   pl.BlockSpec((1, t_rows, cb), lambda b: (b, 0, 0))],
        out_shape=[jax.ShapeDtypeStruct((bs, t_rows, ca), _F32),
                   jax.ShapeDtypeStruct((bs, t_rows, cb), _F32)],
    )(tgt, src, fa, fb)


def _near1(tgt, src, fa):
    bs, t_rows, _ = tgt.shape
    p = src.shape[1]
    ca = fa.shape[2]
    return _pcall(
        _near1_body,
        grid=(bs,),
        in_specs=[pl.BlockSpec((1, t_rows, 3), lambda b: (b, 0, 0)),
                  pl.BlockSpec((1, p, 3), lambda b: (b, 0, 0)),
                  pl.BlockSpec((1, p, ca), lambda b: (b, 0, 0))],
        out_specs=pl.BlockSpec((1, t_rows, ca), lambda b: (b, 0, 0)),
        out_shape=jax.ShapeDtypeStruct((bs, t_rows, ca), _F32),
    )(tgt, src, fa)


def _gmax_body(x_ref, o_ref):
    o_ref[...] = jnp.max(x_ref[...], axis=1)


def _gmax(x):
    bs, _, c = x.shape
    return _pcall(
        _gmax_body,
        out_shape=jax.ShapeDtypeStruct((bs, c), _F32),
    )(x)


def kernel(vertices, rgb_f, dir0, w_rgb, b_rgb, g_rgb, be_rgb,
           w1, b1, d1, g1, be1, w2, b2, d2, g2, be2,
           w3, b3, d3, g3, be3, w4, b4, d4):
    bs, v, _ = vertices.shape

    nb = _knn(vertices, vertices, 10)
    fm0_s = _conv_surface(nb, vertices, dir0, 64, tile=128)

    rgb = _mm(jnp.transpose(rgb_f, (0, 2, 1)), w_rgb.T, b_rgb, relu=True)
    rgb = _bn(rgb, g_rgb, be_rgb, relu=False)
    fm0 = jnp.concatenate([fm0_s, rgb], axis=-1)

    fout1 = _mm(fm0, w1, b1)
    fm1 = _conv_layer(nb, vertices, fout1[:, :, :128], fout1[:, :, 128:],
                      d1, 128, tile=128)
    fm1 = _bn(fm1, g1, be1, relu=True)

    sel1 = jax.random.permutation(jax.random.key(42), v)[:v // 4]
    v1 = vertices[:, sel1, :]
    fmp1 = _pool(v1, vertices, fm1, 4)
    nb1 = _knn(v1, v1, 10)

    fout2 = _mm(fmp1, w2, b2)
    fm2 = _conv_layer(nb1, v1, fout2[:, :, :256], fout2[:, :, 256:],
                      d2, 256, tile=128)
    fm2 = _bn(fm2, g2, be2, relu=True)

    fout3 = _mm(fm2, w3, b3)
    fm3 = _conv_layer(nb1, v1, fout3[:, :, :256], fout3[:, :, 256:],
                      d3, 256, tile=128)
    fm3 = _bn(fm3, g3, be3, relu=True)

    sel2 = jax.random.permutation(jax.random.key(43), v // 4)[:v // 16]
    v2 = v1[:, sel2, :]
    fmp2 = _pool(v2, v1, fm3, 4)
    nb2 = _knn(v2, v2, 8)

    fout4 = _mm(fmp2, w4, b4)
    fm4 = _conv_layer(nb2, v2, fout4[:, :, :512], fout4[:, :, 512:],
                      d4, 512, tile=64)

    fglob = _gmax(fm4)
    fm2u, fm3u = _near2(vertices, v1, fm2, fm3)
    fm4u = _near1(vertices, v2, fm4)

    fg = jnp.broadcast_to(fglob[:, None, :], (bs, v, fglob.shape[-1]))
    feat = jnp.concatenate([fm0, fm1, fm2u, fm3u, fm4u], axis=2)
    fuse = jnp.concatenate([fm0, fm1, fm2u, fm3u, fm4u, fg], axis=2)
    return jnp.transpose(feat, (0, 2, 1)), jnp.transpose(fuse, (0, 2, 1))


# speedup vs baseline: 1.1403x; 1.1403x over previous
import functools

import jax
import jax.numpy as jnp
from jax import lax
from jax.experimental import pallas as pl

_S = 7
_F32 = jnp.float32
_INTERPRET = False


def _pcall(body, **kw):
    return pl.pallas_call(body, interpret=_INTERPRET, **kw)


_EXACT = lax.Precision.HIGHEST


def _gdot3(oh, vals):
    h1 = vals.astype(jnp.bfloat16)
    r1 = vals - h1.astype(_F32)
    h2 = r1.astype(jnp.bfloat16)
    h3 = (r1 - h2.astype(_F32)).astype(jnp.bfloat16)
    ohb = oh.astype(jnp.bfloat16)
    return (jnp.dot(ohb, h1, preferred_element_type=_F32) +
            jnp.dot(ohb, h2, preferred_element_type=_F32) +
            jnp.dot(ohb, h3, preferred_element_type=_F32))


def _gdot(oh, vals):
    hi = vals.astype(jnp.bfloat16)
    lo = (vals - hi.astype(_F32)).astype(jnp.bfloat16)
    ohb = oh.astype(jnp.bfloat16)
    return (jnp.dot(ohb, hi, preferred_element_type=_F32) +
            jnp.dot(ohb, lo, preferred_element_type=_F32))


def _rowsum_sq_bcast(x, t_rows):
    s2 = jnp.sum(x * x, axis=1, keepdims=True)
    ones = jnp.ones((t_rows, 1), _F32)
    return lax.dot_general(ones, s2, (((1,), (1,)), ((), ())),
                           preferred_element_type=_F32, precision=_EXACT)


def _inner_tn(a, b):
    return lax.dot_general(a.astype(jnp.bfloat16), b.astype(jnp.bfloat16),
                           (((1,), (1,)), ((), ())),
                           preferred_element_type=_F32)


def _knn_body(tgt_ref, src_ref, out_ref, *, k):
    tgt = tgt_ref[0]
    src = src_ref[0]
    t_rows = tgt.shape[0]
    v = src.shape[0]
    inner = _inner_tn(tgt, src)
    t2 = jnp.sum(tgt * tgt, axis=1, keepdims=True)
    s2bc = _rowsum_sq_bcast(src, t_rows)
    d = (-2.0 * inner + s2bc) + t2
    viota = lax.broadcasted_iota(jnp.int32, (t_rows, v), 1)
    cur = d
    cols = []
    for j in range(k + 1):
        m = jnp.min(cur, axis=1, keepdims=True)
        am = jnp.min(jnp.where(cur <= m, viota, v), axis=1, keepdims=True)
        cols.append(am)
        cur = jnp.where(viota == am, jnp.inf, cur)
    out_ref[0] = jnp.concatenate(cols[1:], axis=1)


def _knn(tgt, src, k):
    bs, t_rows, _ = tgt.shape
    v = src.shape[1]
    return _pcall(
        functools.partial(_knn_body, k=k),
        grid=(bs,),
        in_specs=[pl.BlockSpec((1, t_rows, 3), lambda b: (b, 0, 0)),
                  pl.BlockSpec((1, v, 3), lambda b: (b, 0, 0))],
        out_specs=pl.BlockSpec((1, t_rows, k), lambda b: (b, 0, 0)),
        out_shape=jax.ShapeDtypeStruct((bs, t_rows, k), jnp.int32),
    )(tgt, src)


def _support_sum(maxed, cout):
    sc = maxed.shape[1]
    r = lax.broadcasted_iota(jnp.int32, (sc, cout), 0)
    c = lax.broadcasted_iota(jnp.int32, (sc, cout), 1)
    smat = (jnp.bitwise_and(r, cout - 1) == c).astype(jnp.bfloat16)
    hi = maxed.astype(jnp.bfloat16)
    lo = (maxed - hi.astype(_F32)).astype(jnp.bfloat16)
    return (jnp.dot(hi, smat, preferred_element_type=_F32) +
            jnp.dot(lo, smat, preferred_element_type=_F32))


def _normed_dirs(dirs):
    nd = jnp.sqrt(jnp.sum(dirs * dirs, axis=0, keepdims=True))
    return dirs / jnp.maximum(nd, 1e-12)


def _neighbor_theta(verts, vt, nb, sdn, j, viota):
    colj = nb[:, j:j + 1]
    oh = (viota == colj).astype(_F32)
    nbrs = _gdot3(oh, verts)
    diff = nbrs - vt
    dn = jnp.sqrt(jnp.sum(diff * diff, axis=1, keepdims=True))
    dirn = diff / jnp.maximum(dn, 1e-12)
    theta = jnp.maximum(
        jnp.dot(dirn.astype(jnp.bfloat16), sdn.astype(jnp.bfloat16),
                preferred_element_type=_F32), 0.0)
    return oh, theta


def _surface_body(verts_ref, vt_ref, nb_ref, dirs_ref, out_ref, *, n, cout):
    verts = verts_ref[0]
    vt = vt_ref[0]
    nb = nb_ref[0]
    sdn = _normed_dirs(dirs_ref[...])
    t_rows = vt.shape[0]
    v = verts.shape[0]
    viota = lax.broadcasted_iota(jnp.int32, (t_rows, v), 1)
    macc = jnp.full((t_rows, _S * cout), -jnp.inf, _F32)
    for j in range(n):
        _, theta = _neighbor_theta(verts, vt, nb, sdn, j, viota)
        macc = jnp.maximum(macc, theta)
    out_ref[0] = jnp.maximum(_support_sum(macc, cout), 0.0)


def _layer_body(verts_ref, vt_ref, nb_ref, dirs_ref, fc_ref, fs_ref, out_ref,
                *, n, cout):
    verts = verts_ref[0]
    vt = vt_ref[0]
    nb = nb_ref[0]
    sdn = _normed_dirs(dirs_ref[...])
    fs_full = fs_ref[0]
    t_rows = vt.shape[0]
    v = verts.shape[0]
    viota = lax.broadcasted_iota(jnp.int32, (t_rows, v), 1)
    macc = jnp.full((t_rows, _S * cout), -jnp.inf, _F32)
    for j in range(n):
        oh, theta = _neighbor_theta(verts, vt, nb, sdn, j, viota)
        fs = _gdot(oh, fs_full)
        macc = jnp.maximum(macc, theta * fs)
    out_ref[0] = fc_ref[0] + _support_sum(macc, cout)


def _conv_surface(nb, vertices, dirs, cout, tile):
    bs, v, n = nb.shape
    nt = v // tile
    return _pcall(
        functools.partial(_surface_body, n=n, cout=cout),
        grid=(bs, nt),
        in_specs=[pl.BlockSpec((1, v, 3), lambda b, t: (b, 0, 0)),
                  pl.BlockSpec((1, tile, 3), lambda b, t: (b, t, 0)),
                  pl.BlockSpec((1, tile, n), lambda b, t: (b, t, 0)),
                  pl.BlockSpec((3, _S * cout), lambda b, t: (0, 0))],
        out_specs=pl.BlockSpec((1, tile, cout), lambda b, t: (b, t, 0)),
        out_shape=jax.ShapeDtypeStruct((bs, v, cout), _F32),
    )(vertices, vertices, nb, dirs)


def _conv_layer(nb, vertices, fc, fs, dirs, cout, tile):
    bs, v, n = nb.shape
    nt = v // tile
    sc = _S * cout
    return _pcall(
        functools.partial(_layer_body, n=n, cout=cout),
        grid=(bs, nt),
        in_specs=[pl.BlockSpec((1, v, 3), lambda b, t: (b, 0, 0)),
                  pl.BlockSpec((1, tile, 3), lambda b, t: (b, t, 0)),
                  pl.BlockSpec((1, tile, n), lambda b, t: (b, t, 0)),
                  pl.BlockSpec((3, sc), lambda b, t: (0, 0)),
                  pl.BlockSpec((1, tile, cout), lambda b, t: (b, t, 0)),
                  pl.BlockSpec((1, v, sc), lambda b, t: (b, 0, 0))],
        out_specs=pl.BlockSpec((1, tile, cout), lambda b, t: (b, t, 0)),
        out_shape=jax.ShapeDtypeStruct((bs, v, cout), _F32),
    )(vertices, vertices, nb, dirs, fc, fs)


def _mm_body(x_ref, w_ref, b_ref, o_ref, *, relu):
    y = jnp.dot(x_ref[0].astype(jnp.bfloat16),
                w_ref[...].astype(jnp.bfloat16),
                preferred_element_type=_F32) + b_ref[...]
    if relu:
        y = jnp.maximum(y, 0.0)
    o_ref[0] = y


def _mm(x, w, b, relu=False):
    bs, m, k = x.shape
    n = w.shape[1]
    return _pcall(
        functools.partial(_mm_body, relu=relu),
        grid=(bs,),
        in_specs=[pl.BlockSpec((1, m, k), lambda b_: (b_, 0, 0)),
                  pl.BlockSpec((k, n), lambda b_: (0, 0)),
                  pl.BlockSpec((1, n), lambda b_: (0, 0))],
        out_specs=pl.BlockSpec((1, m, n), lambda b_: (b_, 0, 0)),
        out_shape=jax.ShapeDtypeStruct((bs, m, n), _F32),
    )(x, w, b[None, :])


def _bn_body(x_ref, g_ref, b_ref, o_ref, *, n_rows, relu):
    x = x_ref[...]
    m = jnp.sum(x, axis=0, keepdims=True) / n_rows
    xc = x - m
    var = jnp.sum(xc * xc, axis=0, keepdims=True) / n_rows
    y = g_ref[...] * xc / jnp.sqrt(var + 1e-5) + b_ref[...]
    if relu:
        y = jnp.maximum(y, 0.0)
    o_ref[...] = y


def _bn(x, g, b, relu):
    bs, v, c = x.shape
    x2 = x.reshape(bs * v, c)
    y = _pcall(
        functools.partial(_bn_body, n_rows=float(bs * v), relu=relu),
        out_shape=jax.ShapeDtypeStruct((bs * v, c), _F32),
    )(x2, g[None, :], b[None, :])
    return y.reshape(bs, v, c)


def _pool_body(nb_ref, fm_ref, out_ref, *, k):
    nb = nb_ref[0]
    fm = fm_ref[0]
    t_rows = nb.shape[0]
    v = fm.shape[0]
    viota = lax.broadcasted_iota(jnp.int32, (t_rows, v), 1)
    pooled = jnp.full((t_rows, fm.shape[1]), -jnp.inf, _F32)
    for j in range(k):
        oh = (viota == nb[:, j:j + 1]).astype(_F32)
        pooled = jnp.maximum(pooled, _gdot(oh, fm))
    out_ref[0] = pooled


def _pool(nbsel, fm, k):
    bs, t_rows, _ = nbsel.shape
    v, c = fm.shape[1], fm.shape[2]
    return _pcall(
        functools.partial(_pool_body, k=k),
        grid=(bs,),
        in_specs=[pl.BlockSpec((1, t_rows, k), lambda b: (b, 0, 0)),
                  pl.BlockSpec((1, v, c), lambda b: (b, 0, 0))],
        out_specs=pl.BlockSpec((1, t_rows, c), lambda b: (b, 0, 0)),
        out_shape=jax.ShapeDtypeStruct((bs, t_rows, c), _F32),
    )(nbsel, fm)


def _nearest_oh(tgt, src):
    t_rows = tgt.shape[0]
    p = src.shape[0]
    inner = _inner_tn(tgt, src)
    t2 = jnp.sum(tgt * tgt, axis=1, keepdims=True)
    s2bc = _rowsum_sq_bcast(src, t_rows)
    d = (s2bc + t2) - 2.0 * inner
    piota = lax.broadcasted_iota(jnp.int32, (t_rows, p), 1)
    m = jnp.min(d, axis=1, keepdims=True)
    am = jnp.min(jnp.where(d <= m, piota, p), axis=1, keepdims=True)
    return (piota == am).astype(_F32)


def _near2_body(tgt_ref, src_ref, fa_ref, fb_ref, oa_ref, ob_ref):
    oh = _nearest_oh(tgt_ref[0], src_ref[0])
    oa_ref[0] = _gdot(oh, fa_ref[0])
    ob_ref[0] = _gdot(oh, fb_ref[0])


def _near1_body(tgt_ref, src_ref, fa_ref, oa_ref):
    oh = _nearest_oh(tgt_ref[0], src_ref[0])
    oa_ref[0] = _gdot(oh, fa_ref[0])


def _near2(tgt, src, fa, fb):
    bs, t_rows, _ = tgt.shape
    p = src.shape[1]
    ca, cb = fa.shape[2], fb.shape[2]
    return _pcall(
        _near2_body,
        grid=(bs,),
        in_specs=[pl.BlockSpec((1, t_rows, 3), lambda b: (b, 0, 0)),
                  pl.BlockSpec((1, p, 3), lambda b: (b, 0, 0)),
                  pl.BlockSpec((1, p, ca), lambda b: (b, 0, 0)),
                  pl.BlockSpec((1, p, cb), lambda b: (b, 0, 0))],
        out_specs=[pl.BlockSpec((1, t_rows, ca), lambda b: (b, 0, 0)),
                   pl.BlockSpec((1, t_rows, cb), lambda b: (b, 0, 0))],
        out_shape=[jax.ShapeDtypeStruct((bs, t_rows, ca), _F32),
                   jax.ShapeDtypeStruct((bs, t_rows, cb), _F32)],
    )(tgt, src, fa, fb)


def _near1(tgt, src, fa):
    bs, t_rows, _ = tgt.shape
    p = src.shape[1]
    ca = fa.shape[2]
    return _pcall(
        _near1_body,
        grid=(bs,),
        in_specs=[pl.BlockSpec((1, t_rows, 3), lambda b: (b, 0, 0)),
                  pl.BlockSpec((1, p, 3), lambda b: (b, 0, 0)),
                  pl.BlockSpec((1, p, ca), lambda b: (b, 0, 0))],
        out_specs=pl.BlockSpec((1, t_rows, ca), lambda b: (b, 0, 0)),
        out_shape=jax.ShapeDtypeStruct((bs, t_rows, ca), _F32),
    )(tgt, src, fa)


def _gmax_body(x_ref, o_ref):
    o_ref[...] = jnp.max(x_ref[...], axis=1)


def _gmax(x):
    bs, _, c = x.shape
    return _pcall(
        _gmax_body,
        out_shape=jax.ShapeDtypeStruct((bs, c), _F32),
    )(x)


def kernel(vertices, rgb_f, dir0, w_rgb, b_rgb, g_rgb, be_rgb,
           w1, b1, d1, g1, be1, w2, b2, d2, g2, be2,
           w3, b3, d3, g3, be3, w4, b4, d4):
    bs, v, _ = vertices.shape

    nb = _knn(vertices, vertices, 10)
    fm0_s = _conv_surface(nb, vertices, dir0, 64, tile=128)

    rgb = _mm(jnp.transpose(rgb_f, (0, 2, 1)), w_rgb.T, b_rgb, relu=True)
    rgb = _bn(rgb, g_rgb, be_rgb, relu=False)
    fm0 = jnp.concatenate([fm0_s, rgb], axis=-1)

    fout1 = _mm(fm0, w1, b1)
    fm1 = _conv_layer(nb, vertices, fout1[:, :, :128], fout1[:, :, 128:],
                      d1, 128, tile=128)
    fm1 = _bn(fm1, g1, be1, relu=True)

    sel1 = jax.random.permutation(jax.random.key(42), v)[:v // 4]
    v1 = vertices[:, sel1, :]
    fmp1 = _pool(nb[:, sel1, :4], fm1, 4)
    nb1 = _knn(v1, v1, 10)

    fout2 = _mm(fmp1, w2, b2)
    fm2 = _conv_layer(nb1, v1, fout2[:, :, :256], fout2[:, :, 256:],
                      d2, 256, tile=128)
    fm2 = _bn(fm2, g2, be2, relu=True)

    fout3 = _mm(fm2, w3, b3)
    fm3 = _conv_layer(nb1, v1, fout3[:, :, :256], fout3[:, :, 256:],
                      d3, 256, tile=128)
    fm3 = _bn(fm3, g3, be3, relu=True)

    sel2 = jax.random.permutation(jax.random.key(43), v // 4)[:v // 16]
    v2 = v1[:, sel2, :]
    fmp2 = _pool(nb1[:, sel2, :4], fm3, 4)
    nb2 = _knn(v2, v2, 8)

    fout4 = _mm(fmp2, w4, b4)
    fm4 = _conv_layer(nb2, v2, fout4[:, :, :512], fout4[:, :, 512:],
                      d4, 512, tile=64)

    fglob = _gmax(fm4)
    fm2u, fm3u = _near2(vertices, v1, fm2, fm3)
    fm4u = _near1(vertices, v2, fm4)

    fg = jnp.broadcast_to(fglob[:, None, :], (bs, v, fglob.shape[-1]))
    feat = jnp.concatenate([fm0, fm1, fm2u, fm3u, fm4u], axis=2)
    fuse = jnp.concatenate([fm0, fm1, fm2u, fm3u, fm4u, fg], axis=2)
    return jnp.transpose(feat, (0, 2, 1)), jnp.transpose(fuse, (0, 2, 1))


# speedup vs baseline: 1.3492x; 1.1832x over previous
import functools

import jax
import jax.numpy as jnp
from jax import lax
from jax.experimental import pallas as pl

_S = 7
_F32 = jnp.float32
_INTERPRET = False


def _pcall(body, **kw):
    return pl.pallas_call(body, interpret=_INTERPRET, **kw)


_EXACT = lax.Precision.HIGHEST


def _gdot3(oh, vals):
    h1 = vals.astype(jnp.bfloat16)
    r1 = vals - h1.astype(_F32)
    h2 = r1.astype(jnp.bfloat16)
    h3 = (r1 - h2.astype(_F32)).astype(jnp.bfloat16)
    ohb = oh.astype(jnp.bfloat16)
    return (jnp.dot(ohb, h1, preferred_element_type=_F32) +
            jnp.dot(ohb, h2, preferred_element_type=_F32) +
            jnp.dot(ohb, h3, preferred_element_type=_F32))


def _gdot(oh, vals):
    hi = vals.astype(jnp.bfloat16)
    lo = (vals - hi.astype(_F32)).astype(jnp.bfloat16)
    ohb = oh.astype(jnp.bfloat16)
    return (jnp.dot(ohb, hi, preferred_element_type=_F32) +
            jnp.dot(ohb, lo, preferred_element_type=_F32))


def _rowsum_sq_bcast(x, t_rows):
    s2 = jnp.sum(x * x, axis=1, keepdims=True)
    ones = jnp.ones((t_rows, 1), _F32)
    return lax.dot_general(ones, s2, (((1,), (1,)), ((), ())),
                           preferred_element_type=_F32, precision=_EXACT)


def _inner_tn(a, b):
    return lax.dot_general(a.astype(jnp.bfloat16), b.astype(jnp.bfloat16),
                           (((1,), (1,)), ((), ())),
                           preferred_element_type=_F32)


def _knn_body(tgt_ref, src_ref, out_ref, *, k):
    tgt = tgt_ref[0]
    src = src_ref[0]
    t_rows = tgt.shape[0]
    v = src.shape[0]
    inner = _inner_tn(tgt, src)
    t2 = jnp.sum(tgt * tgt, axis=1, keepdims=True)
    s2bc = _rowsum_sq_bcast(src, t_rows)
    d = (-2.0 * inner + s2bc) + t2
    viota = lax.broadcasted_iota(jnp.int32, (t_rows, v), 1)
    cur = d
    cols = []
    for j in range(k + 1):
        m = jnp.min(cur, axis=1, keepdims=True)
        am = jnp.min(jnp.where(cur <= m, viota, v), axis=1, keepdims=True)
        cols.append(am)
        cur = jnp.where(viota == am, jnp.inf, cur)
    out_ref[0] = jnp.concatenate(cols[1:], axis=1)


def _knn(tgt, src, k):
    bs, t_rows, _ = tgt.shape
    v = src.shape[1]
    return _pcall(
        functools.partial(_knn_body, k=k),
        grid=(bs,),
        in_specs=[pl.BlockSpec((1, t_rows, 3), lambda b: (b, 0, 0)),
                  pl.BlockSpec((1, v, 3), lambda b: (b, 0, 0))],
        out_specs=pl.BlockSpec((1, t_rows, k), lambda b: (b, 0, 0)),
        out_shape=jax.ShapeDtypeStruct((bs, t_rows, k), jnp.int32),
    )(tgt, src)


def _support_sum(maxed, cout):
    sc = maxed.shape[1]
    r = lax.broadcasted_iota(jnp.int32, (sc, cout), 0)
    c = lax.broadcasted_iota(jnp.int32, (sc, cout), 1)
    smat = (jnp.bitwise_and(r, cout - 1) == c).astype(jnp.bfloat16)
    hi = maxed.astype(jnp.bfloat16)
    lo = (maxed - hi.astype(_F32)).astype(jnp.bfloat16)
    return (jnp.dot(hi, smat, preferred_element_type=_F32) +
            jnp.dot(lo, smat, preferred_element_type=_F32))


def _normed_dirs(dirs):
    nd = jnp.sqrt(jnp.sum(dirs * dirs, axis=0, keepdims=True))
    return dirs / jnp.maximum(nd, 1e-12)


def _neighbor_dirn(verts, vt, nb, j, viota):
    colj = nb[:, j:j + 1]
    oh = (viota == colj).astype(_F32)
    nbrs = _gdot3(oh, verts)
    diff = nbrs - vt
    dn = jnp.sqrt(jnp.sum(diff * diff, axis=1, keepdims=True))
    return oh, diff / jnp.maximum(dn, 1e-12)


def _theta(dirn, sdn):
    return jnp.maximum(
        jnp.dot(dirn.astype(jnp.bfloat16), sdn.astype(jnp.bfloat16),
                preferred_element_type=_F32), 0.0)


def _surface_body(verts_ref, vt_ref, nb_ref, dirs_ref, out_ref, dout_ref,
                  *, n, cout):
    verts = verts_ref[0]
    vt = vt_ref[0]
    nb = nb_ref[0]
    sdn = _normed_dirs(dirs_ref[...])
    t_rows = vt.shape[0]
    v = verts.shape[0]
    viota = lax.broadcasted_iota(jnp.int32, (t_rows, v), 1)
    macc = jnp.full((t_rows, _S * cout), -jnp.inf, _F32)
    for j in range(n):
        _, dirn = _neighbor_dirn(verts, vt, nb, j, viota)
        dout_ref[0, j] = dirn
        macc = jnp.maximum(macc, _theta(dirn, sdn))
    out_ref[0] = jnp.maximum(_support_sum(macc, cout), 0.0)


def _layer_prod_body(verts_ref, vt_ref, nb_ref, dirs_ref, fc_ref, fs_ref,
                     out_ref, dout_ref, *, n, cout):
    verts = verts_ref[0]
    vt = vt_ref[0]
    nb = nb_ref[0]
    sdn = _normed_dirs(dirs_ref[...])
    fs_full = fs_ref[0]
    t_rows = vt.shape[0]
    v = verts.shape[0]
    viota = lax.broadcasted_iota(jnp.int32, (t_rows, v), 1)
    macc = jnp.full((t_rows, _S * cout), -jnp.inf, _F32)
    for j in range(n):
        oh, dirn = _neighbor_dirn(verts, vt, nb, j, viota)
        dout_ref[0, j] = dirn
        macc = jnp.maximum(macc, _theta(dirn, sdn) * _gdot(oh, fs_full))
    out_ref[0] = fc_ref[0] + _support_sum(macc, cout)


def _layer_cons_body(nb_ref, dirs_ref, fc_ref, fs_ref, dirn_ref, out_ref,
                     *, n, cout):
    nb = nb_ref[0]
    sdn = _normed_dirs(dirs_ref[...])
    fs_full = fs_ref[0]
    t_rows = nb.shape[0]
    v = fs_full.shape[0]
    viota = lax.broadcasted_iota(jnp.int32, (t_rows, v), 1)
    macc = jnp.full((t_rows, _S * cout), -jnp.inf, _F32)
    for j in range(n):
        oh = (viota == nb[:, j:j + 1]).astype(_F32)
        macc = jnp.maximum(macc, _theta(dirn_ref[0, j], sdn) *
                           _gdot(oh, fs_full))
    out_ref[0] = fc_ref[0] + _support_sum(macc, cout)


def _conv_surface(nb, vertices, dirs, cout, tile):
    bs, v, n = nb.shape
    nt = v // tile
    return _pcall(
        functools.partial(_surface_body, n=n, cout=cout),
        grid=(bs, nt),
        in_specs=[pl.BlockSpec((1, v, 3), lambda b, t: (b, 0, 0)),
                  pl.BlockSpec((1, tile, 3), lambda b, t: (b, t, 0)),
                  pl.BlockSpec((1, tile, n), lambda b, t: (b, t, 0)),
                  pl.BlockSpec((3, _S * cout), lambda b, t: (0, 0))],
        out_specs=[pl.BlockSpec((1, tile, cout), lambda b, t: (b, t, 0)),
                   pl.BlockSpec((1, n, tile, 3), lambda b, t: (b, 0, t, 0))],
        out_shape=[jax.ShapeDtypeStruct((bs, v, cout), _F32),
                   jax.ShapeDtypeStruct((bs, n, v, 3), _F32)],
    )(vertices, vertices, nb, dirs)


def _conv_layer_prod(nb, vertices, fc, fs, dirs, cout, tile):
    bs, v, n = nb.shape
    nt = v // tile
    sc = _S * cout
    return _pcall(
        functools.partial(_layer_prod_body, n=n, cout=cout),
        grid=(bs, nt),
        in_specs=[pl.BlockSpec((1, v, 3), lambda b, t: (b, 0, 0)),
                  pl.BlockSpec((1, tile, 3), lambda b, t: (b, t, 0)),
                  pl.BlockSpec((1, tile, n), lambda b, t: (b, t, 0)),
                  pl.BlockSpec((3, sc), lambda b, t: (0, 0)),
                  pl.BlockSpec((1, tile, cout), lambda b, t: (b, t, 0)),
                  pl.BlockSpec((1, v, sc), lambda b, t: (b, 0, 0))],
        out_specs=[pl.BlockSpec((1, tile, cout), lambda b, t: (b, t, 0)),
                   pl.BlockSpec((1, n, tile, 3), lambda b, t: (b, 0, t, 0))],
        out_shape=[jax.ShapeDtypeStruct((bs, v, cout), _F32),
                   jax.ShapeDtypeStruct((bs, n, v, 3), _F32)],
    )(vertices, vertices, nb, dirs, fc, fs)


def _conv_layer_cons(nb, dirn, fc, fs, dirs, cout, tile):
    bs, v, n = nb.shape
    nt = v // tile
    sc = _S * cout
    return _pcall(
        functools.partial(_layer_cons_body, n=n, cout=cout),
        grid=(bs, nt),
        in_specs=[pl.BlockSpec((1, tile, n), lambda b, t: (b, t, 0)),
                  pl.BlockSpec((3, sc), lambda b, t: (0, 0)),
                  pl.BlockSpec((1, tile, cout), lambda b, t: (b, t, 0)),
                  pl.BlockSpec((1, v, sc), lambda b, t: (b, 0, 0)),
                  pl.BlockSpec((1, n, tile, 3), lambda b, t: (b, 0, t, 0))],
        out_specs=pl.BlockSpec((1, tile, cout), lambda b, t: (b, t, 0)),
        out_shape=jax.ShapeDtypeStruct((bs, v, cout), _F32),
    )(nb, dirs, fc, fs, dirn)


def _conv_layer(nb, vertices, fc, fs, dirs, cout, tile):
    bs, v, n = nb.shape
    nt = v // tile
    sc = _S * cout
    return _pcall(
        functools.partial(_layer_prod_body, n=n, cout=cout),
        grid=(bs, nt),
        in_specs=[pl.BlockSpec((1, v, 3), lambda b, t: (b, 0, 0)),
                  pl.BlockSpec((1, tile, 3), lambda b, t: (b, t, 0)),
                  pl.BlockSpec((1, tile, n), lambda b, t: (b, t, 0)),
                  pl.BlockSpec((3, sc), lambda b, t: (0, 0)),
                  pl.BlockSpec((1, tile, cout), lambda b, t: (b, t, 0)),
                  pl.BlockSpec((1, v, sc), lambda b, t: (b, 0, 0))],
        out_specs=[pl.BlockSpec((1, tile, cout), lambda b, t: (b, t, 0)),
                   pl.BlockSpec((1, n, tile, 3), lambda b, t: (b, 0, t, 0))],
        out_shape=[jax.ShapeDtypeStruct((bs, v, cout), _F32),
                   jax.ShapeDtypeStruct((bs, n, v, 3), _F32)],
    )(vertices, vertices, nb, dirs, fc, fs)[0]


def _mm_body(x_ref, w_ref, b_ref, o_ref, *, relu):
    y = jnp.dot(x_ref[0].astype(jnp.bfloat16),
                w_ref[...].astype(jnp.bfloat16),
                preferred_element_type=_F32) + b_ref[...]
    if relu:
        y = jnp.maximum(y, 0.0)
    o_ref[0] = y


def _mm(x, w, b, relu=False):
    bs, m, k = x.shape
    n = w.shape[1]
    return _pcall(
        functools.partial(_mm_body, relu=relu),
        grid=(bs,),
        in_specs=[pl.BlockSpec((1, m, k), lambda b_: (b_, 0, 0)),
                  pl.BlockSpec((k, n), lambda b_: (0, 0)),
                  pl.BlockSpec((1, n), lambda b_: (0, 0))],
        out_specs=pl.BlockSpec((1, m, n), lambda b_: (b_, 0, 0)),
        out_shape=jax.ShapeDtypeStruct((bs, m, n), _F32),
    )(x, w, b[None, :])


def _bn_body(x_ref, g_ref, b_ref, o_ref, *, n_rows, relu):
    x = x_ref[...]
    m = jnp.sum(x, axis=0, keepdims=True) / n_rows
    xc = x - m
    var = jnp.sum(xc * xc, axis=0, keepdims=True) / n_rows
    y = g_ref[...] * xc / jnp.sqrt(var + 1e-5) + b_ref[...]
    if relu:
        y = jnp.maximum(y, 0.0)
    o_ref[...] = y


def _bn(x, g, b, relu):
    bs, v, c = x.shape
    x2 = x.reshape(bs * v, c)
    y = _pcall(
        functools.partial(_bn_body, n_rows=float(bs * v), relu=relu),
        out_shape=jax.ShapeDtypeStruct((bs * v, c), _F32),
    )(x2, g[None, :], b[None, :])
    return y.reshape(bs, v, c)


def _pool_body(nb_ref, fm_ref, out_ref, *, k):
    nb = nb_ref[0]
    fm = fm_ref[0]
    t_rows = nb.shape[0]
    v = fm.shape[0]
    viota = lax.broadcasted_iota(jnp.int32, (t_rows, v), 1)
    pooled = jnp.full((t_rows, fm.shape[1]), -jnp.inf, _F32)
    for j in range(k):
        oh = (viota == nb[:, j:j + 1]).astype(_F32)
        pooled = jnp.maximum(pooled, _gdot(oh, fm))
    out_ref[0] = pooled


def _pool(nbsel, fm, k):
    bs, t_rows, _ = nbsel.shape
    v, c = fm.shape[1], fm.shape[2]
    return _pcall(
        functools.partial(_pool_body, k=k),
        grid=(bs,),
        in_specs=[pl.BlockSpec((1, t_rows, k), lambda b: (b, 0, 0)),
                  pl.BlockSpec((1, v, c), lambda b: (b, 0, 0))],
        out_specs=pl.BlockSpec((1, t_rows, c), lambda b: (b, 0, 0)),
        out_shape=jax.ShapeDtypeStruct((bs, t_rows, c), _F32),
    )(nbsel, fm)


def _nearest_oh(tgt, src):
    t_rows = tgt.shape[0]
    p = src.shape[0]
    inner = _inner_tn(tgt, src)
    t2 = jnp.sum(tgt * tgt, axis=1, keepdims=True)
    s2bc = _rowsum_sq_bcast(src, t_rows)
    d = (s2bc + t2) - 2.0 * inner
    piota = lax.broadcasted_iota(jnp.int32, (t_rows, p), 1)
    m = jnp.min(d, axis=1, keepdims=True)
    am = jnp.min(jnp.where(d <= m, piota, p), axis=1, keepdims=True)
    return (piota == am).astype(_F32)


def _near2_body(tgt_ref, src_ref, fa_ref, fb_ref, oa_ref, ob_ref):
    oh = _nearest_oh(tgt_ref[0], src_ref[0])
    oa_ref[0] = _gdot(oh, fa_ref[0])
    ob_ref[0] = _gdot(oh, fb_ref[0])


def _near1_body(tgt_ref, src_ref, fa_ref, oa_ref):
    oh = _nearest_oh(tgt_ref[0], src_ref[0])
    oa_ref[0] = _gdot(oh, fa_ref[0])


def _near2(tgt, src, fa, fb):
    bs, t_rows, _ = tgt.shape
    p = src.shape[1]
    ca, cb = fa.shape[2], fb.shape[2]
    return _pcall(
        _near2_body,
        grid=(bs,),
        in_specs=[pl.BlockSpec((1, t_rows, 3), lambda b: (b, 0, 0)),
                  pl.BlockSpec((1, p, 3), lambda b: (b, 0, 0)),
                  pl.BlockSpec((1, p, ca), lambda b: (b, 0, 0)),
                  pl.BlockSpec((1, p, cb), lambda b: (b, 0, 0))],
        out_specs=[pl.BlockSpec((1, t_rows, ca), lambda b: (b, 0, 0)),
                   pl.BlockSpec((1, t_rows, cb), lambda b: (b, 0, 0))],
        out_shape=[jax.ShapeDtypeStruct((bs, t_rows, ca), _F32),
                   jax.ShapeDtypeStruct((bs, t_rows, cb), _F32)],
    )(tgt, src, fa, fb)


def _near1(tgt, src, fa):
    bs, t_rows, _ = tgt.shape
    p = src.shape[1]
    ca = fa.shape[2]
    return _pcall(
        _near1_body,
        grid=(bs,),
        in_specs=[pl.BlockSpec((1, t_rows, 3), lambda b: (b, 0, 0)),
                  pl.BlockSpec((1, p, 3), lambda b: (b, 0, 0)),
                  pl.BlockSpec((1, p, ca), lambda b: (b, 0, 0))],
        out_specs=pl.BlockSpec((1, t_rows, ca), lambda b: (b, 0, 0)),
        out_shape=jax.ShapeDtypeStruct((bs, t_rows, ca), _F32),
    )(tgt, src, fa)


def _gmax_body(x_ref, o_ref):
    o_ref[...] = jnp.max(x_ref[...], axis=1)


def _gmax(x):
    bs, _, c = x.shape
    return _pcall(
        _gmax_body,
        out_shape=jax.ShapeDtypeStruct((bs, c), _F32),
    )(x)


def kernel(vertices, rgb_f, dir0, w_rgb, b_rgb, g_rgb, be_rgb,
           w1, b1, d1, g1, be1, w2, b2, d2, g2, be2,
           w3, b3, d3, g3, be3, w4, b4, d4):
    bs, v, _ = vertices.shape

    nb = _knn(vertices, vertices, 10)
    fm0_s, dirn0 = _conv_surface(nb, vertices, dir0, 64, tile=128)

    rgb = _mm(jnp.transpose(rgb_f, (0, 2, 1)), w_rgb.T, b_rgb, relu=True)
    rgb = _bn(rgb, g_rgb, be_rgb, relu=False)
    fm0 = jnp.concatenate([fm0_s, rgb], axis=-1)

    fout1 = _mm(fm0, w1, b1)
    fm1 = _conv_layer_cons(nb, dirn0, fout1[:, :, :128], fout1[:, :, 128:],
                           d1, 128, tile=128)
    fm1 = _bn(fm1, g1, be1, relu=True)

    sel1 = jax.random.permutation(jax.random.key(42), v)[:v // 4]
    v1 = vertices[:, sel1, :]
    fmp1 = _pool(nb[:, sel1, :4], fm1, 4)
    nb1 = _knn(v1, v1, 10)

    fout2 = _mm(fmp1, w2, b2)
    fm2, dirn1 = _conv_layer_prod(nb1, v1, fout2[:, :, :256],
                                  fout2[:, :, 256:], d2, 256, tile=128)
    fm2 = _bn(fm2, g2, be2, relu=True)

    fout3 = _mm(fm2, w3, b3)
    fm3 = _conv_layer_cons(nb1, dirn1, fout3[:, :, :256], fout3[:, :, 256:],
                           d3, 256, tile=128)
    fm3 = _bn(fm3, g3, be3, relu=True)

    sel2 = jax.random.permutation(jax.random.key(43), v // 4)[:v // 16]
    v2 = v1[:, sel2, :]
    fmp2 = _pool(nb1[:, sel2, :4], fm3, 4)
    nb2 = _knn(v2, v2, 8)

    fout4 = _mm(fmp2, w4, b4)
    fm4 = _conv_layer(nb2, v2, fout4[:, :, :512], fout4[:, :, 512:],
                      d4, 512, tile=64)

    fglob = _gmax(fm4)
    fm2u, fm3u = _near2(vertices, v1, fm2, fm3)
    fm4u = _near1(vertices, v2, fm4)

    fg = jnp.broadcast_to(fglob[:, None, :], (bs, v, fglob.shape[-1]))
    feat = jnp.concatenate([fm0, fm1, fm2u, fm3u, fm4u], axis=2)
    fuse = jnp.concatenate([fm0, fm1, fm2u, fm3u, fm4u, fg], axis=2)
    return jnp.transpose(feat, (0, 2, 1)), jnp.transpose(fuse, (0, 2, 1))
